# baseline (speedup 1.0000x reference)
import functools

import jax
import jax.numpy as jnp
from jax import lax
from jax.experimental import pallas as pl
from jax.experimental.pallas import tpu as pltpu
from jax.experimental.pallas import tpu_sc as plsc

N = 320000
D = 128
S = 10000
NC = 2
NS = 16
NW = NC * NS
CHUNK = 80
NCH_TOT = N // CHUNK
KPT = NCH_TOT // NW
NPAIR = (KPT - 1) // 2
STRIPE = 624
TAIL = S - NS * STRIPE

RB = 32000


def _scores_body(x_ref, w1_ref, b1_ref, w2_ref, b2_ref, ex_ref):
    h = jnp.tanh(
        jnp.dot(x_ref[...], w1_ref[...], preferred_element_type=jnp.float32)
        + b1_ref[...]
    )
    s = jnp.dot(h, w2_ref[...], preferred_element_type=jnp.float32) + b2_ref[...]
    ex_ref[...] = jnp.exp(s).reshape(1, RB // 128, 128)


def _scores(x, W1, b1, W2, b2):
    return pl.pallas_call(
        _scores_body,
        grid=(N // RB,),
        in_specs=[
            pl.BlockSpec((RB, D), lambda i: (i, 0)),
            pl.BlockSpec((D, D // 2), lambda i: (0, 0)),
            pl.BlockSpec((1, D // 2), lambda i: (0, 0)),
            pl.BlockSpec((D // 2, 1), lambda i: (0, 0)),
            pl.BlockSpec((1, 1), lambda i: (0, 0)),
        ],
        out_specs=pl.BlockSpec((1, RB // 128, 128), lambda i: (i, 0, 0)),
        out_shape=jax.ShapeDtypeStruct((N // RB, RB // 128, 128), jnp.float32),
    )(x, W1, b1.reshape(1, -1), W2, b2.reshape(1, 1))


def _pool_body(x_hbm, ex_hbm, ids_hbm, acc_out, den_out,
               xbuf, wbuf, exbuf, wexbuf, idbuf, sidbuf, denstage,
               acc_sp, den_sp,
               insem0, insem1, outsem0, outsem1):
    c = lax.axis_index("c")
    sid = lax.axis_index("s")
    wid = c * NS + sid
    r0 = sid * STRIPE
    insem = (insem0, insem1)
    outsem = (outsem0, outsem1)

    def _zero_row(i, _):
        for j in range(D // 16):
            wbuf[0, i, pl.ds(j * 16, 16)] = jnp.zeros((16,), jnp.float32)
        return 0

    lax.fori_loop(0, CHUNK, _zero_row, 0)
    for g in range(STRIPE // 16):
        denstage[pl.ds(g * 16, 16)] = jnp.zeros((16,), jnp.float32)

    z2d = wbuf.at[0]
    nfull = STRIPE // CHUNK
    for part in range(nfull):
        pltpu.sync_copy(z2d, acc_sp.at[pl.ds(r0 + part * CHUNK, CHUNK)])
    pltpu.sync_copy(z2d.at[pl.ds(0, STRIPE - nfull * CHUNK)],
                    acc_sp.at[pl.ds(r0 + nfull * CHUNK,
                                    STRIPE - nfull * CHUNK)])
    pltpu.sync_copy(denstage, den_sp.at[pl.ds(r0, STRIPE)])

    @pl.when(sid == 0)
    def _zero_tail():
        pltpu.sync_copy(z2d.at[pl.ds(0, TAIL)],
                        acc_sp.at[pl.ds(NS * STRIPE, TAIL)])
        pltpu.sync_copy(denstage.at[pl.ds(0, TAIL)],
                        den_sp.at[pl.ds(NS * STRIPE, TAIL)])

    plsc.subcore_barrier()

    def _issue_in(b, k):
        ci = k * NW + wid
        r = ci * CHUNK
        pltpu.async_copy(x_hbm.at[pl.ds(r, CHUNK)], xbuf.at[b], insem[b])
        pltpu.async_copy(ex_hbm.at[pl.ds(r, CHUNK)], exbuf.at[b], insem[b])
        pltpu.async_copy(ids_hbm.at[pl.ds(r, CHUNK)], idbuf.at[b, 0],
                         insem[b])

    def _wait_in(b, k):
        ci = k * NW + wid
        r = ci * CHUNK
        pltpu.make_async_copy(x_hbm.at[pl.ds(r, CHUNK)], xbuf.at[b],
                              insem[b]).wait()
        pltpu.make_async_copy(ex_hbm.at[pl.ds(r, CHUNK)], exbuf.at[b],
                              insem[b]).wait()
        pltpu.make_async_copy(ids_hbm.at[pl.ds(r, CHUNK)], idbuf.at[b, 0],
                              insem[b]).wait()

    def _compute(b):
        def _grp(g, _):
            iv = idbuf[b, 0, pl.ds(g * 16, 16)]
            sidbuf[b, 0, pl.ds(g * 16, 16)] = iv
            ev = exbuf[b, pl.ds(g * 16, 16)]
            wexbuf[b, pl.ds(g * 16, 16)] = ev
            for q in range(16):
                i = g * 16 + q
                e = ev[q]
                for j in range(D // 16):
                    wbuf[b, i, pl.ds(j * 16, 16)] = (
                        xbuf[b, i, pl.ds(j * 16, 16)] * e)
            return 0

        lax.fori_loop(0, CHUNK // 16, _grp, 0)

    def _issue_out(b):
        pltpu.async_copy(wbuf.at[b], acc_sp.at[sidbuf.at[b, 0]],
                         outsem[b], add=True)
        pltpu.async_copy(wexbuf.at[b], den_sp.at[sidbuf.at[b, 0]],
                         outsem[b], add=True)

    def _wait_out(b):
        pltpu.make_async_copy(wbuf.at[b], acc_sp.at[sidbuf.at[b, 0]],
                              outsem[b]).wait()
        pltpu.make_async_copy(wexbuf.at[b], den_sp.at[sidbuf.at[b, 0]],
                              outsem[b]).wait()

    _issue_in(0, 0)
    _issue_in(1, 1)

    def _pair(p, _):
        k0 = 2 * p
        for b in (0, 1):
            k = k0 + b
            _wait_in(b, k)

            @pl.when(p >= 1)
            def _():
                _wait_out(b)

            _compute(b)
            _issue_out(b)

            @pl.when(k + 2 <= KPT - 1)
            def _():
                _issue_in(b, k + 2)
        return 0

    lax.fori_loop(0, NPAIR, _pair, 0)

    _wait_in(0, KPT - 1)
    _wait_out(0)
    _compute(0)
    _issue_out(0)
    _wait_out(1)
    _wait_out(0)

    plsc.subcore_barrier()
    pltpu.sync_copy(acc_sp.at[pl.ds(r0, STRIPE)],
                    acc_out.at[c, pl.ds(r0, STRIPE)])
    pltpu.sync_copy(den_sp.at[pl.ds(r0, STRIPE)], denstage)
    pltpu.sync_copy(denstage, den_out.at[pl.ds(c * S + r0, STRIPE)])

    @pl.when(sid == 0)
    def _copy_tail():
        pltpu.sync_copy(acc_sp.at[pl.ds(NS * STRIPE, TAIL)],
                        acc_out.at[c, pl.ds(NS * STRIPE, TAIL)])
        pltpu.sync_copy(den_sp.at[pl.ds(NS * STRIPE, TAIL)],
                        denstage.at[pl.ds(0, TAIL)])
        pltpu.sync_copy(denstage.at[pl.ds(0, TAIL)],
                        den_out.at[pl.ds(c * S + NS * STRIPE, TAIL)])


@functools.cache
def _pool():
    mesh = plsc.VectorSubcoreMesh(
        core_axis_name="c", subcore_axis_name="s",
        num_cores=NC, num_subcores=NS,
    )
    return pl.kernel(
        _pool_body,
        out_type=[
            jax.ShapeDtypeStruct((NC, S, D), jnp.float32),
            jax.ShapeDtypeStruct((NC * S,), jnp.float32),
        ],
        mesh=mesh,
        scratch_types=[
            pltpu.VMEM((2, CHUNK, D), jnp.float32),
            pltpu.VMEM((2, CHUNK, D), jnp.float32),
            pltpu.VMEM((2, CHUNK), jnp.float32),
            pltpu.VMEM((2, CHUNK), jnp.float32),
            pltpu.VMEM((2, 1, CHUNK), jnp.int32),
            pltpu.VMEM((2, 1, CHUNK), jnp.int32),
            pltpu.VMEM((STRIPE,), jnp.float32),
            pltpu.VMEM_SHARED((S, D), jnp.float32),
            pltpu.VMEM_SHARED((S,), jnp.float32),
            pltpu.SemaphoreType.DMA,
            pltpu.SemaphoreType.DMA,
            pltpu.SemaphoreType.DMA,
            pltpu.SemaphoreType.DMA,
        ],
    )


def _combine_body(acc_ref, den_ref, out_ref):
    den_all = den_ref[...]
    den = (den_all[0:S] + den_all[S:2 * S])[:, None]
    w = acc_ref[...][0] + acc_ref[...][1]
    safe = jnp.where(den > 0, den, 1.0)
    out_ref[...] = jnp.where(den > 0, w / safe, 0.0)


def _combine(acc, den):
    return pl.pallas_call(
        _combine_body,
        out_shape=jax.ShapeDtypeStruct((S, D), jnp.float32),
    )(acc, den)


def kernel(x, batch, W1, b1, W2, b2):
    ex = _scores(x, W1, b1, W2, b2)
    acc, den = _pool()(x, ex.reshape(N), batch)
    return _combine(acc, den)
